# baseline (speedup 1.0000x reference)
import functools

import jax
import jax.numpy as jnp
from jax import lax
from jax.experimental import pallas as pl
from jax.experimental.pallas import tpu as pltpu
from jax.experimental.pallas import tpu_sc as plsc


@functools.lru_cache(maxsize=None)
def _build_gather(B, V, D):
    info = plsc.get_sparse_core_info()
    NC, NS = info.num_cores, info.num_subcores
    NW = NC * NS
    assert B % (8 * NW) == 0
    b_per_w = B // NW
    mesh = plsc.VectorSubcoreMesh(core_axis_name="c", subcore_axis_name="s")

    @functools.partial(
        pl.kernel,
        mesh=mesh,
        out_type=jax.ShapeDtypeStruct((B, D), jnp.float32),
        scratch_types=[
            pltpu.VMEM((2, b_per_w // 2), jnp.int32),
            pltpu.VMEM((2, b_per_w // 2, D), jnp.float32),
            pltpu.SemaphoreType.DMA,
            pltpu.SemaphoreType.DMA,
            pltpu.SemaphoreType.DMA,
            pltpu.SemaphoreType.DMA,
        ],
    )
    def gather_kernel(idx_hbm, table_hbm, out_hbm, idx_v, rows_v,
                      g0, g1, w0, w1):
        h = b_per_w // 2
        wid = lax.axis_index("s") * NC + lax.axis_index("c")
        base = wid * b_per_w
        pltpu.sync_copy(idx_hbm.at[pl.ds(base, h)], idx_v.at[0])
        pltpu.sync_copy(idx_hbm.at[pl.ds(base + h, h)], idx_v.at[1])
        ga = pltpu.async_copy(table_hbm.at[idx_v.at[0]], rows_v.at[0], g0)
        gb = pltpu.async_copy(table_hbm.at[idx_v.at[1]], rows_v.at[1], g1)
        ga.wait()
        wa = pltpu.async_copy(rows_v.at[0], out_hbm.at[pl.ds(base, h)], w0)
        gb.wait()
        wb = pltpu.async_copy(rows_v.at[1], out_hbm.at[pl.ds(base + h, h)], w1)
        wa.wait()
        wb.wait()

    return gather_kernel


def kernel(indices, table):
    B, = indices.shape
    V, D = table.shape
    idx = indices.astype(jnp.int32)
    return _build_gather(B, V, D)(idx, table)
